# baseline (speedup 1.0000x reference)
import functools

import jax
import jax.numpy as jnp
from jax import lax
from jax.experimental import pallas as pl
from jax.experimental.pallas import tpu as pltpu
from jax.experimental.pallas import tpu_sc as plsc

S = 7
C = 20
BATCH = 4096
F = C + 10
CELLS = S * S
NFEAT = CELLS * F

NC = 2
NS = 16
NW = NC * NS
IMGS_PER_W = BATCH // NW
CH = 16
N_CH = IMGS_PER_W // CH
IN_CH = CH * NFEAT
OUT_CH = CH * CELLS * 6

_mesh = plsc.VectorSubcoreMesh(core_axis_name="c", subcore_axis_name="s")


@functools.partial(
    pl.kernel,
    mesh=_mesh,
    compiler_params=pltpu.CompilerParams(needs_layout_passes=False),
    out_type=(
        jax.ShapeDtypeStruct((BATCH * CELLS * 6,), jnp.float32),
        jax.ShapeDtypeStruct((BATCH * CELLS * 6,), jnp.float32),
        jax.ShapeDtypeStruct((BATCH,), jnp.int32),
        jax.ShapeDtypeStruct((BATCH,), jnp.int32),
    ),
    scratch_types=[
        pltpu.VMEM((IN_CH,), jnp.float32),
        pltpu.VMEM((IN_CH,), jnp.float32),
        pltpu.VMEM((OUT_CH,), jnp.float32),
        pltpu.VMEM((OUT_CH,), jnp.float32),
        pltpu.VMEM((CH,), jnp.int32),
        pltpu.VMEM((CH,), jnp.int32),
        pltpu.SemaphoreType.DMA,
        pltpu.SemaphoreType.DMA,
    ],
)
def _sc_map_kernel(pred_hbm, tgt_hbm, mp_hbm, mt_hbm, pc_hbm, tc_hbm,
                   in0, in1, out0, out1, cnt0, cnt1, in_sem, out_sem):
    wid = lax.axis_index("s") * NC + lax.axis_index("c")
    lane = lax.iota(jnp.int32, 16)
    gat_base = lane * NFEAT
    sct_base = lane * (CELLS * 6)
    inv_s = jnp.float32(1.0 / S)

    def process_chunk(in_ref, out_ref, cntout_ref, thresh):
        def cell_body(i, cnt):
            base = gat_base + i * F

            def g(f):
                return plsc.load_gather(in_ref, [base + f])

            m = g(0)
            label = jnp.zeros((16,), jnp.float32)
            for k in range(1, C):
                p = g(k)
                gt = p > m
                m = jnp.where(gt, p, m)
                label = jnp.where(gt, jnp.float32(k), label)

            conf1 = g(C)
            conf2 = g(C + 5)
            best = conf2 > conf1
            bb0 = jnp.where(best, g(C + 6), g(C + 1))
            bb1 = jnp.where(best, g(C + 7), g(C + 2))
            bb2 = jnp.where(best, g(C + 8), g(C + 3))
            bb3 = jnp.where(best, g(C + 9), g(C + 4))

            col = (i % S).astype(jnp.float32)
            row = (i // S).astype(jnp.float32)
            cx = (bb0 + col) * inv_s
            cy = (bb1 + row) * inv_s
            w2 = bb2 * inv_s * 0.5
            h2 = bb3 * inv_s * 0.5
            conf = jnp.maximum(conf1, conf2)
            mask = conf > thresh

            outs = (cx - w2, cy - h2, cx + w2, cy + h2, conf, label)
            ob = sct_base + i * 6
            zero = jnp.zeros((16,), jnp.float32)
            for k in range(6):
                plsc.store_scatter(out_ref, [ob + k],
                                   jnp.where(mask, outs[k], zero))
            return cnt + jnp.where(mask, 1, 0).astype(jnp.int32)

        cnt = lax.fori_loop(0, CELLS, cell_body, jnp.zeros((16,), jnp.int32))
        cntout_ref[...] = cnt

    def run_tensor(src, dst, cnt_hbm, thresh):
        base_img = wid * IMGS_PER_W
        bufs = ((in0, out0, cnt0), (in1, out1, cnt1))

        def start_in(ci, b):
            pltpu.async_copy(src.at[pl.ds((base_img + ci * CH) * NFEAT, IN_CH)],
                             bufs[b][0], in_sem)

        def start_out(ci, b):
            o = (base_img + ci * CH) * CELLS * 6
            pltpu.async_copy(bufs[b][1], dst.at[pl.ds(o, OUT_CH)], out_sem)
            pltpu.async_copy(bufs[b][2],
                             cnt_hbm.at[pl.ds(base_img + ci * CH, CH)], out_sem)

        def wait_in(b):
            pltpu.make_async_copy(src.at[pl.ds(0, IN_CH)],
                                  bufs[b][0], in_sem).wait()

        def wait_out(b):
            pltpu.make_async_copy(bufs[b][1],
                                  dst.at[pl.ds(0, OUT_CH)], out_sem).wait()
            pltpu.make_async_copy(bufs[b][2],
                                  cnt_hbm.at[pl.ds(0, CH)], out_sem).wait()

        def half(ci, b, first_pair):
            wait_in(b)

            @pl.when(jnp.logical_not(first_pair))
            def _():
                wait_out(b)

            process_chunk(bufs[b][0], bufs[b][1], bufs[b][2], thresh)
            start_out(ci, b)

            @pl.when(ci + 2 < N_CH)
            def _():
                start_in(ci + 2, b)

        start_in(0, 0)
        start_in(1, 1)

        def pair_body(cp, _):
            ci0 = cp * 2
            first = cp == 0
            half(ci0, 0, first)
            half(ci0 + 1, 1, first)
            return 0

        lax.fori_loop(0, N_CH // 2, pair_body, 0)
        wait_out(0)
        wait_out(1)

    run_tensor(pred_hbm, mp_hbm, pc_hbm, jnp.float32(0.1))
    run_tensor(tgt_hbm, mt_hbm, tc_hbm, jnp.float32(0.5))


def kernel(predictions, targets):
    p = predictions.reshape(-1)
    t = targets.reshape(-1)
    mp, mt, pc, tc = _sc_map_kernel(p, t)
    return (mp.reshape(BATCH, CELLS, 6),
            mt.reshape(BATCH, CELLS, 6),
            pc, tc)
